# Initial kernel scaffold; baseline (speedup 1.0000x reference)
import functools

import jax
import jax.numpy as jnp
from jax import lax
from jax.experimental import pallas as pl
from jax.experimental.pallas import tpu as pltpu
from jax.experimental.pallas import tpu_sc as plsc

N = 10000
E = 320000
D = 128

NC = 2
NS = 16
NW = NC * NS

CHUNK = 128
NCHUNKS = E // CHUNK
KMAX = (NCHUNKS + NW - 1) // NW

ROWS_T = 624
ZROWS = 208


def _sc_body(feat_hbm, src_hbm, dst_hbm, sums_out, cnts_out,
             sums_sh, cnts_sh, srcv, dstv, rows, onesv, zrow, zcnt, sem):
    cid = lax.axis_index("c")
    sid = lax.axis_index("s")
    wid = sid * NC + cid

    zeros16 = jnp.zeros((16,), jnp.float32)
    ones16 = jnp.ones((16,), jnp.float32)

    def zr_row(r, _):
        def zr_col(j, _):
            zrow[r, pl.ds(pl.multiple_of(j * 16, 16), 16)] = zeros16
            return 0
        return lax.fori_loop(0, D // 16, zr_col, 0)

    lax.fori_loop(0, ZROWS, zr_row, 0)

    def zc(i, _):
        zcnt[pl.ds(pl.multiple_of(i * 16, 16), 16)] = zeros16
        return 0

    lax.fori_loop(0, (ROWS_T + 16) // 16, zc, 0)

    for j in range(CHUNK // 16):
        onesv[pl.ds(j * 16, 16)] = ones16

    row0 = sid * ROWS_T
    for b in range(3):
        pltpu.sync_copy(zrow, sums_sh.at[pl.ds(row0 + b * ZROWS, ZROWS)])
    pltpu.sync_copy(zcnt.at[pl.ds(0, ROWS_T)], cnts_sh.at[pl.ds(row0, ROWS_T)])

    @pl.when(sid == NS - 1)
    def _():
        pltpu.sync_copy(zrow.at[pl.ds(0, 16)],
                        sums_sh.at[pl.ds(N - 16, 16)])
        pltpu.sync_copy(zcnt.at[pl.ds(0, 16)], cnts_sh.at[pl.ds(N - 16, 16)])

    plsc.subcore_barrier()

    def step(k, _):
        c = k * NW + wid

        @pl.when(c < NCHUNKS)
        def _():
            base = pl.multiple_of(c * CHUNK, CHUNK)
            pltpu.sync_copy(src_hbm.at[pl.ds(base, CHUNK)], srcv)
            pltpu.sync_copy(dst_hbm.at[pl.ds(base, CHUNK)], dstv)
            pltpu.async_copy(feat_hbm.at[srcv], rows, sem).wait()
            pltpu.sync_copy(rows, sums_sh.at[dstv], add=True)
            pltpu.sync_copy(onesv, cnts_sh.at[dstv], add=True)

        return 0

    lax.fori_loop(0, KMAX, step, 0)

    plsc.subcore_barrier()

    out0 = cid * N + row0
    pltpu.sync_copy(sums_sh.at[pl.ds(row0, ROWS_T)],
                    sums_out.at[pl.ds(out0, ROWS_T)])
    pltpu.sync_copy(cnts_sh.at[pl.ds(row0, ROWS_T)],
                    cnts_out.at[pl.ds(out0, ROWS_T)])

    @pl.when(sid == NS - 1)
    def _():
        pltpu.sync_copy(sums_sh.at[pl.ds(N - 16, 16)],
                        sums_out.at[pl.ds(cid * N + N - 16, 16)])
        pltpu.sync_copy(cnts_sh.at[pl.ds(N - 16, 16)],
                        cnts_out.at[pl.ds(cid * N + N - 16, 16)])


_sc_scatter = functools.partial(
    pl.kernel,
    out_type=(
        jax.ShapeDtypeStruct((NC * N, D), jnp.float32),
        jax.ShapeDtypeStruct((NC * N,), jnp.float32),
    ),
    mesh=plsc.VectorSubcoreMesh(core_axis_name="c", subcore_axis_name="s"),
    scratch_types=(
        pltpu.VMEM_SHARED((N, D), jnp.float32),
        pltpu.VMEM_SHARED((N,), jnp.float32),
        pltpu.VMEM((CHUNK,), jnp.int32),
        pltpu.VMEM((CHUNK,), jnp.int32),
        pltpu.VMEM((CHUNK, D), jnp.float32),
        pltpu.VMEM((CHUNK,), jnp.float32),
        pltpu.VMEM((ZROWS, D), jnp.float32),
        pltpu.VMEM((ROWS_T + 16,), jnp.float32),
        pltpu.SemaphoreType.DMA,
    ),
)(_sc_body)


BLK = 1000


def _tc_body(f_ref, w_ref, s0_ref, s1_ref, c0_ref, c1_ref, o_ref):
    w = w_ref[...]
    s = s0_ref[...] + s1_ref[...]
    cnt = c0_ref[...] + c1_ref[...]
    mean = s * (1.0 / jnp.maximum(cnt, 1.0))
    nodes = jnp.dot(f_ref[...], w, preferred_element_type=jnp.float32)
    agg = jnp.dot(mean, w, preferred_element_type=jnp.float32)
    o_ref[:, :D] = jnp.maximum(nodes, 0.0)
    o_ref[:, D:] = jnp.maximum(agg, 0.0)


def _tc_dense(features, weight, sums2, cnts2):
    grid = N // BLK
    return pl.pallas_call(
        _tc_body,
        grid=(grid,),
        in_specs=[
            pl.BlockSpec((BLK, D), lambda i: (i, 0)),
            pl.BlockSpec((D, D), lambda i: (0, 0)),
            pl.BlockSpec((BLK, D), lambda i: (i, 0)),
            pl.BlockSpec((BLK, D), lambda i: (N // BLK + i, 0)),
            pl.BlockSpec((BLK, 1), lambda i: (i, 0)),
            pl.BlockSpec((BLK, 1), lambda i: (N // BLK + i, 0)),
        ],
        out_specs=pl.BlockSpec((BLK, 2 * D), lambda i: (i, 0)),
        out_shape=jax.ShapeDtypeStruct((N, 2 * D), jnp.float32),
    )(features, weight, sums2, sums2, cnts2, cnts2)


def kernel(features, edges, weight):
    edges = edges.astype(jnp.int32)
    dst = edges[0]
    src = edges[1]
    sums2, cnts2 = _sc_scatter(features, src, dst)
    return _tc_dense(features, weight, sums2, cnts2.reshape(NC * N, 1))


# speedup vs baseline: 7.4379x; 7.4379x over previous
import functools

import jax
import jax.numpy as jnp
from jax import lax
from jax.experimental import pallas as pl
from jax.experimental.pallas import tpu as pltpu
from jax.experimental.pallas import tpu_sc as plsc

N = 10000
E = 320000
D = 128

NC = 2
NS = 16
NW = NC * NS

CHUNK = 128
NCHUNKS = E // CHUNK
KMAX = (NCHUNKS + NW - 1) // NW

ROWS_T = 624
ZROWS = 208


def _sc_body(feat_hbm, src_hbm, dst_hbm, sums_out, cnts_out,
             sums_sh, cnts_sh, srcv, dstv, rows, onesv, zrow, zcnt, sem):
    cid = lax.axis_index("c")
    sid = lax.axis_index("s")
    wid = sid * NC + cid

    zeros16 = jnp.zeros((16,), jnp.float32)
    ones16 = jnp.ones((16,), jnp.float32)

    def zr_row(r, _):
        def zr_col(j, _):
            zrow[r, pl.ds(pl.multiple_of(j * 16, 16), 16)] = zeros16
            return 0
        return lax.fori_loop(0, D // 16, zr_col, 0)

    lax.fori_loop(0, ZROWS, zr_row, 0)

    def zc(i, _):
        zcnt[pl.ds(pl.multiple_of(i * 16, 16), 16)] = zeros16
        return 0

    lax.fori_loop(0, (ROWS_T + 16) // 16, zc, 0)

    for j in range(CHUNK // 16):
        onesv[pl.ds(j * 16, 16)] = ones16

    row0 = sid * ROWS_T
    for b in range(3):
        pltpu.sync_copy(zrow, sums_sh.at[pl.ds(row0 + b * ZROWS, ZROWS)])
    pltpu.sync_copy(zcnt.at[pl.ds(0, ROWS_T)], cnts_sh.at[pl.ds(row0, ROWS_T)])

    @pl.when(sid == NS - 1)
    def _():
        pltpu.sync_copy(zrow.at[pl.ds(0, 16)],
                        sums_sh.at[pl.ds(N - 16, 16)])
        pltpu.sync_copy(zcnt.at[pl.ds(0, 16)], cnts_sh.at[pl.ds(N - 16, 16)])

    plsc.subcore_barrier()

    def step(k, _):
        c = k * NW + wid

        @pl.when(c < NCHUNKS)
        def _():
            base = pl.multiple_of(c * CHUNK, CHUNK)
            pltpu.sync_copy(src_hbm.at[pl.ds(base, CHUNK)], srcv)
            pltpu.sync_copy(dst_hbm.at[pl.ds(base, CHUNK)], dstv)
            pltpu.async_copy(feat_hbm.at[srcv], rows, sem).wait()
            pltpu.sync_copy(rows, sums_sh.at[dstv], add=True)
            pltpu.sync_copy(onesv, cnts_sh.at[dstv], add=True)

        return 0

    lax.fori_loop(0, KMAX, step, 0)

    plsc.subcore_barrier()

    out0 = cid * N + row0
    for b in range(3):
        pltpu.sync_copy(sums_sh.at[pl.ds(row0 + b * ZROWS, ZROWS)], zrow)
        pltpu.sync_copy(zrow, sums_out.at[pl.ds(out0 + b * ZROWS, ZROWS)])
    pltpu.sync_copy(cnts_sh.at[pl.ds(row0, ROWS_T)], zcnt.at[pl.ds(0, ROWS_T)])
    pltpu.sync_copy(zcnt.at[pl.ds(0, ROWS_T)], cnts_out.at[pl.ds(out0, ROWS_T)])

    @pl.when(sid == NS - 1)
    def _():
        pltpu.sync_copy(sums_sh.at[pl.ds(N - 16, 16)], zrow.at[pl.ds(0, 16)])
        pltpu.sync_copy(zrow.at[pl.ds(0, 16)],
                        sums_out.at[pl.ds(cid * N + N - 16, 16)])
        pltpu.sync_copy(cnts_sh.at[pl.ds(N - 16, 16)], zcnt.at[pl.ds(0, 16)])
        pltpu.sync_copy(zcnt.at[pl.ds(0, 16)],
                        cnts_out.at[pl.ds(cid * N + N - 16, 16)])


_sc_scatter = functools.partial(
    pl.kernel,
    out_type=(
        jax.ShapeDtypeStruct((NC * N, D), jnp.float32),
        jax.ShapeDtypeStruct((NC * N,), jnp.float32),
    ),
    mesh=plsc.VectorSubcoreMesh(core_axis_name="c", subcore_axis_name="s"),
    scratch_types=(
        pltpu.VMEM_SHARED((N, D), jnp.float32),
        pltpu.VMEM_SHARED((N,), jnp.float32),
        pltpu.VMEM((CHUNK,), jnp.int32),
        pltpu.VMEM((CHUNK,), jnp.int32),
        pltpu.VMEM((CHUNK, D), jnp.float32),
        pltpu.VMEM((CHUNK,), jnp.float32),
        pltpu.VMEM((ZROWS, D), jnp.float32),
        pltpu.VMEM((ROWS_T + 16,), jnp.float32),
        pltpu.SemaphoreType.DMA,
    ),
)(_sc_body)


BLK = 1000


def _tc_body(f_ref, w_ref, s0_ref, s1_ref, c0_ref, c1_ref, o_ref):
    w = w_ref[...]
    s = s0_ref[...] + s1_ref[...]
    cnt = c0_ref[...] + c1_ref[...]
    mean = s * (1.0 / jnp.maximum(cnt, 1.0))
    nodes = jnp.dot(f_ref[...], w, preferred_element_type=jnp.float32)
    agg = jnp.dot(mean, w, preferred_element_type=jnp.float32)
    o_ref[:, :D] = jnp.maximum(nodes, 0.0)
    o_ref[:, D:] = jnp.maximum(agg, 0.0)


def _tc_dense(features, weight, sums2, cnts2):
    grid = N // BLK
    return pl.pallas_call(
        _tc_body,
        grid=(grid,),
        in_specs=[
            pl.BlockSpec((BLK, D), lambda i: (i, 0)),
            pl.BlockSpec((D, D), lambda i: (0, 0)),
            pl.BlockSpec((BLK, D), lambda i: (i, 0)),
            pl.BlockSpec((BLK, D), lambda i: (N // BLK + i, 0)),
            pl.BlockSpec((BLK, 1), lambda i: (i, 0)),
            pl.BlockSpec((BLK, 1), lambda i: (N // BLK + i, 0)),
        ],
        out_specs=pl.BlockSpec((BLK, 2 * D), lambda i: (i, 0)),
        out_shape=jax.ShapeDtypeStruct((N, 2 * D), jnp.float32),
    )(features, weight, sums2, sums2, cnts2, cnts2)


def kernel(features, edges, weight):
    edges = edges.astype(jnp.int32)
    dst = edges[0]
    src = edges[1]
    sums2, cnts2 = _sc_scatter(features, src, dst)
    return _tc_dense(features, weight, sums2, cnts2.reshape(NC * N, 1))
